# baseline (speedup 1.0000x reference)
import functools

import jax
import jax.numpy as jnp
from jax import lax
from jax.experimental import pallas as pl
from jax.experimental.pallas import tpu as pltpu
from jax.experimental.pallas import tpu_sc as plsc


def _sc_gather(table, idx4, n_rows):
    NW, H, K, C = idx4.shape
    E = table.shape[1]
    half_rows = K * C
    NC = 2

    mesh = plsc.VectorSubcoreMesh(core_axis_name="c", subcore_axis_name="s")

    @functools.partial(
        pl.kernel,
        mesh=mesh,
        out_type=jax.ShapeDtypeStruct((n_rows, E), jnp.float32),
        scratch_types=[
            pltpu.VMEM((H, K, C), jnp.int32),
            pltpu.VMEM((half_rows, E), jnp.float32),
            pltpu.SemaphoreType.DMA,
        ],
    )
    def gather_kernel(table_hbm, idx_hbm, out_hbm, idx_v, rows_v, sem):
        wid = lax.axis_index("s") * NC + lax.axis_index("c")
        pltpu.sync_copy(idx_hbm.at[wid], idx_v)
        for h in range(H):
            cps = [
                pltpu.async_copy(
                    table_hbm.at[idx_v.at[h, j]],
                    rows_v.at[pl.ds(j * C, C)],
                    sem,
                )
                for j in range(K)
            ]
            for cp in cps:
                cp.wait()
            pltpu.sync_copy(
                rows_v,
                out_hbm.at[pl.ds((wid * H + h) * half_rows, half_rows)],
            )

    return gather_kernel(table, idx4)


def _pool(rows3, E):
    B, L, EP = rows3.shape
    BB = 512

    def body(r_ref, x_ref):
        r = r_ref[...][:, :, :E]
        ss = jnp.sum(r * r, axis=-1, keepdims=True)
        norms = jnp.sqrt(ss)
        scale = jnp.minimum(1.0, 1.0 / jnp.maximum(norms, 1e-12))
        x_ref[...] = jnp.mean(r * scale, axis=1)

    return pl.pallas_call(
        body,
        grid=(B // BB,),
        in_specs=[pl.BlockSpec((BB, L, EP), lambda i: (i, 0, 0))],
        out_specs=pl.BlockSpec((BB, E), lambda i: (i, 0)),
        out_shape=jax.ShapeDtypeStruct((B, E), jnp.float32),
    )(rows3)


def _project_t(x, W, b2):
    B, E = x.shape
    V = W.shape[0]
    VB = 1024
    grid = pl.cdiv(V, VB)

    def body(x_ref, w_ref, b_ref, o_ref):
        o_ref[...] = (
            lax.dot_general(
                w_ref[...],
                x_ref[...],
                dimension_numbers=(((1,), (1,)), ((), ())),
                preferred_element_type=jnp.float32,
            )
            + b_ref[...]
        )

    return pl.pallas_call(
        body,
        grid=(grid,),
        in_specs=[
            pl.BlockSpec((B, E), lambda i: (0, 0)),
            pl.BlockSpec((VB, E), lambda i: (i, 0)),
            pl.BlockSpec((VB, 1), lambda i: (i, 0)),
        ],
        out_specs=pl.BlockSpec((VB, B), lambda i: (i, 0)),
        out_shape=jax.ShapeDtypeStruct((V, B), jnp.float32),
    )(x, W, b2)


def _project(x, W, b2):
    B, E = x.shape
    V = W.shape[0]
    TB = 1024
    t_idx = (V - TB) // TB + 1
    VB = 512
    NV = t_idx * TB // VB
    NBUF = 4

    def tail_body(x_ref, w_ref, b_ref, o_ref):
        o_ref[...] = (
            lax.dot_general(
                x_ref[...],
                w_ref[...],
                dimension_numbers=(((1,), (1,)), ((), ())),
                preferred_element_type=jnp.float32,
            )
            + b_ref[...]
        )

    y0 = pl.pallas_call(
        tail_body,
        grid=(1,),
        in_specs=[
            pl.BlockSpec((B, E), lambda i: (0, 0)),
            pl.BlockSpec((TB, E), lambda i: (t_idx, 0)),
            pl.BlockSpec((1, TB), lambda i: (0, t_idx)),
        ],
        out_specs=pl.BlockSpec((B, TB), lambda i: (0, t_idx)),
        out_shape=jax.ShapeDtypeStruct((B, V), jnp.float32),
    )(x, W, b2)

    def body(x_ref, w_ref, b_ref, y_in, o_ref, acc, sems):
        del y_in
        i = pl.program_id(0)
        slot = lax.rem(i, NBUF)

        NSPL = 16
        RB = B // NSPL

        def _copies(j, s):
            return [
                pltpu.make_async_copy(
                    acc.at[s, pl.ds(r * RB, RB), :],
                    o_ref.at[pl.ds(r * RB, RB), pl.ds(j * VB, VB)],
                    sems.at[s],
                )
                for r in range(NSPL)
            ]

        @pl.when(i >= NBUF)
        def _wait_oldest():
            for cp in _copies(i - NBUF, slot):
                cp.wait()

        acc[slot] = jnp.broadcast_to(b_ref[...], (B, VB))

        for cp in _copies(i, slot):
            cp.start()

        @pl.when(i == NV - 1)
        def _drain():
            for d in range(NBUF):
                j = i - (NBUF - 1) + d
                s2 = lax.rem(j, NBUF)
                for cp in _copies(j, s2):
                    cp.wait()

    return pl.pallas_call(
        body,
        grid=(NV,),
        in_specs=[
            pl.BlockSpec((B, E), lambda i: (0, 0)),
            pl.BlockSpec((VB, E), lambda i: (i, 0)),
            pl.BlockSpec((1, VB), lambda i: (0, i)),
            pl.BlockSpec(memory_space=pltpu.HBM),
        ],
        out_specs=pl.BlockSpec(memory_space=pltpu.HBM),
        out_shape=jax.ShapeDtypeStruct((B, V), jnp.float32),
        scratch_shapes=[
            pltpu.VMEM((NBUF, B, VB), jnp.float32),
            pltpu.SemaphoreType.DMA((NBUF,)),
        ],
        input_output_aliases={3: 0},
    )(x, W, b2, y0)


def kernel(inputs_, emb_table, W, b):
    B, L = inputs_.shape
    V, E = emb_table.shape
    EP = 128
    n_rows = B * L
    NW, H, C = 32, 4, 128
    K = n_rows // (NW * H * C)

    table_p = jnp.pad(emb_table, ((0, 0), (0, EP - E)))
    idx4 = inputs_.reshape(NW, H, K, C).astype(jnp.int32)
    rows = _sc_gather(table_p, idx4, n_rows)
    x = _pool(rows.reshape(B, L, EP), E)
    return _project_t(x, W, b.reshape(V, 1)).T


# speedup vs baseline: 1.0840x; 1.0840x over previous
import functools

import jax
import jax.numpy as jnp
from jax import lax
from jax.experimental import pallas as pl
from jax.experimental.pallas import tpu as pltpu
from jax.experimental.pallas import tpu_sc as plsc


def _sc_gather(table, idx4, n_rows):
    NW, H, K, C = idx4.shape
    E = table.shape[1]
    half_rows = K * C
    NC = 2

    mesh = plsc.VectorSubcoreMesh(core_axis_name="c", subcore_axis_name="s")

    @functools.partial(
        pl.kernel,
        mesh=mesh,
        out_type=jax.ShapeDtypeStruct((n_rows, E), jnp.float32),
        scratch_types=[
            pltpu.VMEM((H, K, C), jnp.int32),
            pltpu.VMEM((half_rows, E), jnp.float32),
            pltpu.SemaphoreType.DMA,
        ],
    )
    def gather_kernel(table_hbm, idx_hbm, out_hbm, idx_v, rows_v, sem):
        wid = lax.axis_index("s") * NC + lax.axis_index("c")
        pltpu.sync_copy(idx_hbm.at[wid], idx_v)
        for h in range(H):
            cps = [
                pltpu.async_copy(
                    table_hbm.at[idx_v.at[h, j]],
                    rows_v.at[pl.ds(j * C, C)],
                    sem,
                )
                for j in range(K)
            ]
            for cp in cps:
                cp.wait()
            pltpu.sync_copy(
                rows_v,
                out_hbm.at[pl.ds((wid * H + h) * half_rows, half_rows)],
            )

    return gather_kernel(table, idx4)


def _prescale(emb_table, L, EP):
    V, E = emb_table.shape
    BV = 4000

    def body(t_ref, o_ref):
        t = t_ref[...]
        ss = jnp.sum(t * t, axis=-1, keepdims=True)
        norms = jnp.sqrt(ss)
        scale = jnp.minimum(1.0, 1.0 / jnp.maximum(norms, 1e-12)) * (1.0 / L)
        o_ref[...] = jnp.concatenate(
            [t * scale, jnp.zeros((BV, EP - E), jnp.float32)], axis=1
        )

    return pl.pallas_call(
        body,
        grid=(V // BV,),
        in_specs=[pl.BlockSpec((BV, E), lambda i: (i, 0))],
        out_specs=pl.BlockSpec((BV, EP), lambda i: (i, 0)),
        out_shape=jax.ShapeDtypeStruct((V, EP), jnp.float32),
    )(emb_table)


def _sc_gather_sum(table_p, idx4, B, L):
    NW, CH, K, C = idx4.shape
    EP = table_p.shape[1]
    chunk_rows = K * C
    bpc = chunk_rows // L
    bpw = B // NW
    NC = 2

    mesh = plsc.VectorSubcoreMesh(core_axis_name="c", subcore_axis_name="s")

    @functools.partial(
        pl.kernel,
        mesh=mesh,
        out_type=jax.ShapeDtypeStruct((B, EP), jnp.float32),
        scratch_types=[
            pltpu.VMEM((CH, K, C), jnp.int32),
            pltpu.VMEM((chunk_rows, EP), jnp.float32),
            pltpu.VMEM((bpw, EP), jnp.float32),
            pltpu.SemaphoreType.DMA,
        ],
    )
    def gather_sum_kernel(table_hbm, idx_hbm, out_hbm, idx_v, rows_v, xv, sem):
        wid = lax.axis_index("s") * NC + lax.axis_index("c")
        pltpu.sync_copy(idx_hbm.at[wid], idx_v)
        zero = jnp.zeros((16,), jnp.float32)
        for ch in range(CH):
            cps = [
                pltpu.async_copy(
                    table_hbm.at[idx_v.at[ch, j]],
                    rows_v.at[pl.ds(j * C, C)],
                    sem,
                )
                for j in range(K)
            ]
            for cp in cps:
                cp.wait()

            def batch_body(bb, _):
                base = bb * L

                def l_body(l, carry):
                    a0, a1, a2, a3 = carry
                    r = base + l
                    return (
                        a0 + rows_v[r, 0:16],
                        a1 + rows_v[r, 16:32],
                        a2 + rows_v[r, 32:48],
                        a3 + rows_v[r, 48:64],
                    )

                a0, a1, a2, a3 = lax.fori_loop(
                    0, L, l_body, (zero, zero, zero, zero)
                )
                xb = ch * bpc + bb
                xv[xb, 0:16] = a0
                xv[xb, 16:32] = a1
                xv[xb, 32:48] = a2
                xv[xb, 48:64] = a3
                xv[xb, 64:80] = zero
                return 0

            lax.fori_loop(0, bpc, batch_body, 0)
        pltpu.sync_copy(xv, out_hbm.at[pl.ds(wid * bpw, bpw)])

    return gather_sum_kernel(table_p, idx4)


def _pool(rows3, E):
    B, L, EP = rows3.shape
    BB = 512

    def body(r_ref, x_ref):
        r = r_ref[...][:, :, :E]
        ss = jnp.sum(r * r, axis=-1, keepdims=True)
        norms = jnp.sqrt(ss)
        scale = jnp.minimum(1.0, 1.0 / jnp.maximum(norms, 1e-12))
        x_ref[...] = jnp.mean(r * scale, axis=1)

    return pl.pallas_call(
        body,
        grid=(B // BB,),
        in_specs=[pl.BlockSpec((BB, L, EP), lambda i: (i, 0, 0))],
        out_specs=pl.BlockSpec((BB, E), lambda i: (i, 0)),
        out_shape=jax.ShapeDtypeStruct((B, E), jnp.float32),
    )(rows3)


def _project_t(x, W, b2):
    B, EP = x.shape
    V, E = W.shape
    VB = 1024
    grid = pl.cdiv(V, VB)

    def body(x_ref, w_ref, b_ref, o_ref):
        o_ref[...] = (
            lax.dot_general(
                w_ref[...],
                x_ref[...][:, :E],
                dimension_numbers=(((1,), (1,)), ((), ())),
                preferred_element_type=jnp.float32,
            )
            + b_ref[...]
        )

    return pl.pallas_call(
        body,
        grid=(grid,),
        in_specs=[
            pl.BlockSpec((B, EP), lambda i: (0, 0)),
            pl.BlockSpec((VB, E), lambda i: (i, 0)),
            pl.BlockSpec((VB, 1), lambda i: (i, 0)),
        ],
        out_specs=pl.BlockSpec((VB, B), lambda i: (i, 0)),
        out_shape=jax.ShapeDtypeStruct((V, B), jnp.float32),
    )(x, W, b2)


def _project(x, W, b2):
    B, E = x.shape
    V = W.shape[0]
    TB = 1024
    t_idx = (V - TB) // TB + 1
    VB = 512
    NV = t_idx * TB // VB
    NBUF = 4

    def tail_body(x_ref, w_ref, b_ref, o_ref):
        o_ref[...] = (
            lax.dot_general(
                x_ref[...],
                w_ref[...],
                dimension_numbers=(((1,), (1,)), ((), ())),
                preferred_element_type=jnp.float32,
            )
            + b_ref[...]
        )

    y0 = pl.pallas_call(
        tail_body,
        grid=(1,),
        in_specs=[
            pl.BlockSpec((B, E), lambda i: (0, 0)),
            pl.BlockSpec((TB, E), lambda i: (t_idx, 0)),
            pl.BlockSpec((1, TB), lambda i: (0, t_idx)),
        ],
        out_specs=pl.BlockSpec((B, TB), lambda i: (0, t_idx)),
        out_shape=jax.ShapeDtypeStruct((B, V), jnp.float32),
    )(x, W, b2)

    def body(x_ref, w_ref, b_ref, y_in, o_ref, acc, sems):
        del y_in
        i = pl.program_id(0)
        slot = lax.rem(i, NBUF)

        NSPL = 16
        RB = B // NSPL

        def _copies(j, s):
            return [
                pltpu.make_async_copy(
                    acc.at[s, pl.ds(r * RB, RB), :],
                    o_ref.at[pl.ds(r * RB, RB), pl.ds(j * VB, VB)],
                    sems.at[s],
                )
                for r in range(NSPL)
            ]

        @pl.when(i >= NBUF)
        def _wait_oldest():
            for cp in _copies(i - NBUF, slot):
                cp.wait()

        acc[slot] = jnp.broadcast_to(b_ref[...], (B, VB))

        for cp in _copies(i, slot):
            cp.start()

        @pl.when(i == NV - 1)
        def _drain():
            for d in range(NBUF):
                j = i - (NBUF - 1) + d
                s2 = lax.rem(j, NBUF)
                for cp in _copies(j, s2):
                    cp.wait()

    return pl.pallas_call(
        body,
        grid=(NV,),
        in_specs=[
            pl.BlockSpec((B, E), lambda i: (0, 0)),
            pl.BlockSpec((VB, E), lambda i: (i, 0)),
            pl.BlockSpec((1, VB), lambda i: (0, i)),
            pl.BlockSpec(memory_space=pltpu.HBM),
        ],
        out_specs=pl.BlockSpec(memory_space=pltpu.HBM),
        out_shape=jax.ShapeDtypeStruct((B, V), jnp.float32),
        scratch_shapes=[
            pltpu.VMEM((NBUF, B, VB), jnp.float32),
            pltpu.SemaphoreType.DMA((NBUF,)),
        ],
        input_output_aliases={3: 0},
    )(x, W, b2, y0)


def kernel(inputs_, emb_table, W, b):
    B, L = inputs_.shape
    V, E = emb_table.shape
    EP = 128
    n_rows = B * L
    NW, H, C = 32, 4, 128
    K = n_rows // (NW * H * C)

    table_p = _prescale(emb_table, L, EP)
    idx4 = inputs_.reshape(NW, H, K, C).astype(jnp.int32)
    x = _sc_gather_sum(table_p, idx4, B, L)
    return _project_t(x, W, b.reshape(V, 1)).T


# speedup vs baseline: 1.0929x; 1.0082x over previous
import functools

import jax
import jax.numpy as jnp
from jax import lax
from jax.experimental import pallas as pl
from jax.experimental.pallas import tpu as pltpu
from jax.experimental.pallas import tpu_sc as plsc


def _sc_gather(table, idx4, n_rows):
    NW, H, K, C = idx4.shape
    E = table.shape[1]
    half_rows = K * C
    NC = 2

    mesh = plsc.VectorSubcoreMesh(core_axis_name="c", subcore_axis_name="s")

    @functools.partial(
        pl.kernel,
        mesh=mesh,
        out_type=jax.ShapeDtypeStruct((n_rows, E), jnp.float32),
        scratch_types=[
            pltpu.VMEM((H, K, C), jnp.int32),
            pltpu.VMEM((half_rows, E), jnp.float32),
            pltpu.SemaphoreType.DMA,
        ],
    )
    def gather_kernel(table_hbm, idx_hbm, out_hbm, idx_v, rows_v, sem):
        wid = lax.axis_index("s") * NC + lax.axis_index("c")
        pltpu.sync_copy(idx_hbm.at[wid], idx_v)
        for h in range(H):
            cps = [
                pltpu.async_copy(
                    table_hbm.at[idx_v.at[h, j]],
                    rows_v.at[pl.ds(j * C, C)],
                    sem,
                )
                for j in range(K)
            ]
            for cp in cps:
                cp.wait()
            pltpu.sync_copy(
                rows_v,
                out_hbm.at[pl.ds((wid * H + h) * half_rows, half_rows)],
            )

    return gather_kernel(table, idx4)


def _prescale(emb_table, L, EP):
    V, E = emb_table.shape
    BV = 10000

    def body(t_ref, o_ref):
        t = t_ref[...]
        ss = jnp.sum(t * t, axis=-1, keepdims=True)
        norms = jnp.sqrt(ss)
        scale = jnp.minimum(1.0, 1.0 / jnp.maximum(norms, 1e-12)) * (1.0 / L)
        o_ref[...] = jnp.concatenate(
            [t * scale, jnp.zeros((BV, EP - E), jnp.float32)], axis=1
        )

    return pl.pallas_call(
        body,
        grid=(V // BV,),
        in_specs=[pl.BlockSpec((BV, E), lambda i: (i, 0))],
        out_specs=pl.BlockSpec((BV, EP), lambda i: (i, 0)),
        out_shape=jax.ShapeDtypeStruct((V, EP), jnp.float32),
    )(emb_table)


def _sc_gather_sum(table_p, idx4, B, L):
    NW, CH, K, C = idx4.shape
    EP = table_p.shape[1]
    chunk_rows = K * C
    bpc = chunk_rows // L
    bpw = B // NW
    NC = 2

    mesh = plsc.VectorSubcoreMesh(core_axis_name="c", subcore_axis_name="s")

    @functools.partial(
        pl.kernel,
        mesh=mesh,
        out_type=jax.ShapeDtypeStruct((B, EP), jnp.float32),
        scratch_types=[
            pltpu.VMEM((CH, K, C), jnp.int32),
            pltpu.VMEM((chunk_rows, EP), jnp.float32),
            pltpu.VMEM((bpw, EP), jnp.float32),
            pltpu.SemaphoreType.DMA,
        ],
    )
    def gather_sum_kernel(table_hbm, idx_hbm, out_hbm, idx_v, rows_v, xv, sem):
        wid = lax.axis_index("s") * NC + lax.axis_index("c")
        pltpu.sync_copy(idx_hbm.at[wid], idx_v)
        zero = jnp.zeros((16,), jnp.float32)
        for ch in range(CH):
            cps = [
                pltpu.async_copy(
                    table_hbm.at[idx_v.at[ch, j]],
                    rows_v.at[pl.ds(j * C, C)],
                    sem,
                )
                for j in range(K)
            ]
            for cp in cps:
                cp.wait()

            def batch_body(bb, _):
                base = bb * L

                def l_body(l, carry):
                    a0, a1, a2, a3 = carry
                    r = base + l
                    return (
                        a0 + rows_v[r, 0:16],
                        a1 + rows_v[r, 16:32],
                        a2 + rows_v[r, 32:48],
                        a3 + rows_v[r, 48:64],
                    )

                a0, a1, a2, a3 = lax.fori_loop(
                    0, L, l_body, (zero, zero, zero, zero)
                )
                xb = ch * bpc + bb
                xv[xb, 0:16] = a0
                xv[xb, 16:32] = a1
                xv[xb, 32:48] = a2
                xv[xb, 48:64] = a3
                xv[xb, 64:80] = zero
                return 0

            lax.fori_loop(0, bpc, batch_body, 0)
        pltpu.sync_copy(xv, out_hbm.at[pl.ds(wid * bpw, bpw)])

    return gather_sum_kernel(table_p, idx4)


def _pool(rows3, E):
    B, L, EP = rows3.shape
    BB = 512

    def body(r_ref, x_ref):
        r = r_ref[...][:, :, :E]
        ss = jnp.sum(r * r, axis=-1, keepdims=True)
        norms = jnp.sqrt(ss)
        scale = jnp.minimum(1.0, 1.0 / jnp.maximum(norms, 1e-12))
        x_ref[...] = jnp.mean(r * scale, axis=1)

    return pl.pallas_call(
        body,
        grid=(B // BB,),
        in_specs=[pl.BlockSpec((BB, L, EP), lambda i: (i, 0, 0))],
        out_specs=pl.BlockSpec((BB, E), lambda i: (i, 0)),
        out_shape=jax.ShapeDtypeStruct((B, E), jnp.float32),
    )(rows3)


def _project_t(x, W, b2):
    B, EP = x.shape
    V, E = W.shape
    VB = 512
    grid = pl.cdiv(V, VB)

    def body(x_ref, w_ref, b_ref, o_ref):
        o_ref[...] = (
            lax.dot_general(
                w_ref[...],
                x_ref[...][:, :E],
                dimension_numbers=(((1,), (1,)), ((), ())),
                preferred_element_type=jnp.float32,
            )
            + b_ref[...]
        )

    return pl.pallas_call(
        body,
        grid=(grid,),
        in_specs=[
            pl.BlockSpec((B, EP), lambda i: (0, 0)),
            pl.BlockSpec((VB, E), lambda i: (i, 0)),
            pl.BlockSpec((VB, 1), lambda i: (i, 0)),
        ],
        out_specs=pl.BlockSpec((VB, B), lambda i: (i, 0)),
        out_shape=jax.ShapeDtypeStruct((V, B), jnp.float32),
    )(x, W, b2)


def _project(x, W, b2):
    B, E = x.shape
    V = W.shape[0]
    TB = 1024
    t_idx = (V - TB) // TB + 1
    VB = 512
    NV = t_idx * TB // VB
    NBUF = 4

    def tail_body(x_ref, w_ref, b_ref, o_ref):
        o_ref[...] = (
            lax.dot_general(
                x_ref[...],
                w_ref[...],
                dimension_numbers=(((1,), (1,)), ((), ())),
                preferred_element_type=jnp.float32,
            )
            + b_ref[...]
        )

    y0 = pl.pallas_call(
        tail_body,
        grid=(1,),
        in_specs=[
            pl.BlockSpec((B, E), lambda i: (0, 0)),
            pl.BlockSpec((TB, E), lambda i: (t_idx, 0)),
            pl.BlockSpec((1, TB), lambda i: (0, t_idx)),
        ],
        out_specs=pl.BlockSpec((B, TB), lambda i: (0, t_idx)),
        out_shape=jax.ShapeDtypeStruct((B, V), jnp.float32),
    )(x, W, b2)

    def body(x_ref, w_ref, b_ref, y_in, o_ref, acc, sems):
        del y_in
        i = pl.program_id(0)
        slot = lax.rem(i, NBUF)

        NSPL = 16
        RB = B // NSPL

        def _copies(j, s):
            return [
                pltpu.make_async_copy(
                    acc.at[s, pl.ds(r * RB, RB), :],
                    o_ref.at[pl.ds(r * RB, RB), pl.ds(j * VB, VB)],
                    sems.at[s],
                )
                for r in range(NSPL)
            ]

        @pl.when(i >= NBUF)
        def _wait_oldest():
            for cp in _copies(i - NBUF, slot):
                cp.wait()

        acc[slot] = jnp.broadcast_to(b_ref[...], (B, VB))

        for cp in _copies(i, slot):
            cp.start()

        @pl.when(i == NV - 1)
        def _drain():
            for d in range(NBUF):
                j = i - (NBUF - 1) + d
                s2 = lax.rem(j, NBUF)
                for cp in _copies(j, s2):
                    cp.wait()

    return pl.pallas_call(
        body,
        grid=(NV,),
        in_specs=[
            pl.BlockSpec((B, E), lambda i: (0, 0)),
            pl.BlockSpec((VB, E), lambda i: (i, 0)),
            pl.BlockSpec((1, VB), lambda i: (0, i)),
            pl.BlockSpec(memory_space=pltpu.HBM),
        ],
        out_specs=pl.BlockSpec(memory_space=pltpu.HBM),
        out_shape=jax.ShapeDtypeStruct((B, V), jnp.float32),
        scratch_shapes=[
            pltpu.VMEM((NBUF, B, VB), jnp.float32),
            pltpu.SemaphoreType.DMA((NBUF,)),
        ],
        input_output_aliases={3: 0},
    )(x, W, b2, y0)


def kernel(inputs_, emb_table, W, b):
    B, L = inputs_.shape
    V, E = emb_table.shape
    EP = 128
    n_rows = B * L
    NW, H, C = 32, 4, 128
    K = n_rows // (NW * H * C)

    table_p = _prescale(emb_table, L, EP)
    idx4 = inputs_.reshape(NW, H, K, C).astype(jnp.int32)
    x = _sc_gather_sum(table_p, idx4, B, L)
    return _project_t(x, W, b.reshape(V, 1)).T


# speedup vs baseline: 1.0940x; 1.0011x over previous
import functools

import jax
import jax.numpy as jnp
from jax import lax
from jax.experimental import pallas as pl
from jax.experimental.pallas import tpu as pltpu
from jax.experimental.pallas import tpu_sc as plsc


def _prescale(emb_table, L, EP):
    V, E = emb_table.shape
    BV = 10000

    def body(t_ref, o_ref):
        t = t_ref[...]
        ss = jnp.sum(t * t, axis=-1, keepdims=True)
        norms = jnp.sqrt(ss)
        scale = jnp.minimum(1.0, 1.0 / jnp.maximum(norms, 1e-12)) * (1.0 / L)
        o_ref[...] = jnp.concatenate(
            [t * scale, jnp.zeros((BV, EP - E), jnp.float32)], axis=1
        )

    return pl.pallas_call(
        body,
        grid=(V // BV,),
        in_specs=[pl.BlockSpec((BV, E), lambda i: (i, 0))],
        out_specs=pl.BlockSpec((BV, EP), lambda i: (i, 0)),
        out_shape=jax.ShapeDtypeStruct((V, EP), jnp.float32),
    )(emb_table)


def _sc_gather_sum(table_p, idx4, B, L):
    NW, CH, K, C = idx4.shape
    EP = table_p.shape[1]
    chunk_rows = K * C
    bpc = chunk_rows // L
    bpw = B // NW
    NC = 2

    mesh = plsc.VectorSubcoreMesh(core_axis_name="c", subcore_axis_name="s")

    @functools.partial(
        pl.kernel,
        mesh=mesh,
        out_type=jax.ShapeDtypeStruct((B, EP), jnp.float32),
        scratch_types=[
            pltpu.VMEM((CH, K, C), jnp.int32),
            pltpu.VMEM((chunk_rows, EP), jnp.float32),
            pltpu.VMEM((bpw, EP), jnp.float32),
            pltpu.SemaphoreType.DMA,
        ],
    )
    def gather_sum_kernel(table_hbm, idx_hbm, out_hbm, idx_v, rows_v, xv, sem):
        wid = lax.axis_index("s") * NC + lax.axis_index("c")
        pltpu.sync_copy(idx_hbm.at[wid], idx_v)
        zero = jnp.zeros((16,), jnp.float32)
        for ch in range(CH):
            cps = [
                pltpu.async_copy(
                    table_hbm.at[idx_v.at[ch, j]],
                    rows_v.at[pl.ds(j * C, C)],
                    sem,
                )
                for j in range(K)
            ]
            for cp in cps:
                cp.wait()

            def batch_body(bb, _):
                base = bb * L

                def l_body(l, carry):
                    a0, a1, a2, a3 = carry
                    r = base + l
                    return (
                        a0 + rows_v[r, 0:16],
                        a1 + rows_v[r, 16:32],
                        a2 + rows_v[r, 32:48],
                        a3 + rows_v[r, 48:64],
                    )

                a0, a1, a2, a3 = lax.fori_loop(
                    0, L, l_body, (zero, zero, zero, zero)
                )
                xb = ch * bpc + bb
                xv[xb, 0:16] = a0
                xv[xb, 16:32] = a1
                xv[xb, 32:48] = a2
                xv[xb, 48:64] = a3
                xv[xb, 64:80] = zero
                return 0

            lax.fori_loop(0, bpc, batch_body, 0)
        pltpu.sync_copy(xv, out_hbm.at[pl.ds(wid * bpw, bpw)])

    return gather_sum_kernel(table_p, idx4)


def _project_t(x, W, b2):
    B, EP = x.shape
    V, E = W.shape
    VB = 1024
    grid = pl.cdiv(V, VB)

    def body(x_ref, w_ref, b_ref, o_ref):
        o_ref[...] = (
            lax.dot_general(
                w_ref[...],
                x_ref[...][:, :E],
                dimension_numbers=(((1,), (1,)), ((), ())),
                preferred_element_type=jnp.float32,
            )
            + b_ref[...]
        )

    return pl.pallas_call(
        body,
        grid=(grid,),
        in_specs=[
            pl.BlockSpec((B, EP), lambda i: (0, 0)),
            pl.BlockSpec((VB, E), lambda i: (i, 0)),
            pl.BlockSpec((VB, 1), lambda i: (i, 0)),
        ],
        out_specs=pl.BlockSpec((VB, B), lambda i: (i, 0)),
        out_shape=jax.ShapeDtypeStruct((V, B), jnp.float32),
    )(x, W, b2)


def kernel(inputs_, emb_table, W, b):
    B, L = inputs_.shape
    V, E = emb_table.shape
    EP = 128
    n_rows = B * L
    NW, H, C = 32, 4, 128
    K = n_rows // (NW * H * C)

    table_p = _prescale(emb_table, L, EP)
    idx4 = inputs_.reshape(NW, H, K, C).astype(jnp.int32)
    x = _sc_gather_sum(table_p, idx4, B, L)
    return _project_t(x, W, b.reshape(V, 1)).T
